# Initial kernel scaffold; baseline (speedup 1.0000x reference)
import jax
import jax.numpy as jnp
from jax.experimental import pallas as pl


def kernel(x, edge_index, W1l, b1l, W1r, W2l, b2l, W2r, Wh, bh):
    raise NotImplementedError("write your pallas kernel here")


# speedup vs baseline: 4.4537x; 4.4537x over previous
import functools

import jax
import jax.numpy as jnp
from jax import lax
from jax.experimental import pallas as pl
from jax.experimental.pallas import tpu as pltpu
from jax.experimental.pallas import tpu_sc as plsc

N = 10000
NPAD = 10240
E = 160000
D = 256
DH = 128
NS = 16
CHUNK = 80
NGRP = 5
GCHUNK = 25
RPT = NPAD // NS
SLAB = 80
NSLAB = RPT // SLAB
LANES = 16


def _make_agg(with_counts):
    mesh = plsc.VectorSubcoreMesh(core_axis_name="c", subcore_axis_name="s")
    sums_ty = jax.ShapeDtypeStruct((2 * NPAD, DH), jnp.float32)
    out_type = ([sums_ty, jax.ShapeDtypeStruct((NPAD, DH), jnp.float32)]
                if with_counts else sums_ty)
    scratch = [
        pltpu.VMEM((GCHUNK, CHUNK), jnp.int32),
        pltpu.VMEM((GCHUNK, CHUNK), jnp.int32),
        pltpu.VMEM((CHUNK, DH), jnp.float32),
        pltpu.VMEM_SHARED((NPAD, DH), jnp.float32),
        pltpu.SemaphoreType.DMA,
    ]

    def body(xstack, src4d, dst4d, zrows, *rest):
        if with_counts:
            ones_h, out_s, out_c, src_v, dst_v, rows_v, s_sp, sem = rest
        else:
            out_s, src_v, dst_v, rows_v, s_sp, sem = rest
        cid = lax.axis_index("c")
        sid = lax.axis_index("s")
        base = sid * RPT
        coff = cid * N

        def zero_acc():
            pltpu.sync_copy(zrows, rows_v)

            def zslab(j, c):
                pltpu.sync_copy(rows_v, s_sp.at[pl.ds(base + j * SLAB, SLAB)])
                return c
            lax.fori_loop(0, NSLAB, zslab, 0)

        def writeback(out_ref, row0):
            def wslab(j, c):
                pltpu.sync_copy(s_sp.at[pl.ds(base + j * SLAB, SLAB)], rows_v)
                pltpu.sync_copy(
                    rows_v, out_ref.at[pl.ds(row0 + base + j * SLAB, SLAB)])
                return c
            lax.fori_loop(0, NSLAB, wslab, 0)

        zero_acc()
        plsc.subcore_barrier()

        def group(g, cg):
            pltpu.sync_copy(src4d.at[sid, g], src_v)
            pltpu.sync_copy(dst4d.at[sid, g], dst_v)

            def adjust(i, c):
                for k in range(CHUNK // LANES):
                    sl = pl.ds(k * LANES, LANES)
                    src_v[i, sl] = src_v[i, sl] + coff
                return c
            lax.fori_loop(0, GCHUNK, adjust, 0)

            def step(i, c):
                pltpu.async_copy(xstack.at[src_v.at[i]], rows_v, sem).wait()
                pltpu.sync_copy(rows_v, s_sp.at[dst_v.at[i]], add=True)
                return c
            lax.fori_loop(0, GCHUNK, step, 0)
            return cg
        lax.fori_loop(0, NGRP, group, 0)

        plsc.subcore_barrier()

        writeback(out_s, cid * NPAD)

        if with_counts:
            zero_acc()
            plsc.subcore_barrier()
            pltpu.sync_copy(ones_h, rows_v)

            def cgroup(g, cg):
                pltpu.sync_copy(dst4d.at[sid, g], dst_v)

                def cstep(i, c):
                    pltpu.sync_copy(rows_v, s_sp.at[dst_v.at[i]], add=True)
                    return c
                lax.fori_loop(0, GCHUNK, cstep, 0)
                return cg
            lax.fori_loop(0, NGRP, cgroup, 0)

            plsc.subcore_barrier()
            writeback(out_c, 0)

    return functools.partial(
        pl.kernel, mesh=mesh, out_type=out_type, scratch_types=scratch
    )(body)


_agg_counts = _make_agg(True)
_agg_plain = _make_agg(False)


BLK = 1000


def _tc1_body(c_ref, s0_ref, s1_ref, x_ref, wl_ref, wr_ref, b_ref,
              o0_ref, o1_ref):
    inv = 1.0 / jnp.maximum(c_ref[...][:, :1], 1.0)
    agg = jnp.concatenate([s0_ref[...], s1_ref[...]], axis=1) * inv
    h = jnp.dot(agg, wl_ref[...], preferred_element_type=jnp.float32)
    h += jnp.dot(x_ref[...], wr_ref[...], preferred_element_type=jnp.float32)
    h += b_ref[...]
    h = jnp.maximum(h, 0.0)
    o0_ref[...] = h[:, :DH]
    o1_ref[...] = h[:, DH:]


def _tc2_body(c_ref, s0_ref, s1_ref, h0_ref, h1_ref, wl_ref, wr_ref, b_ref,
              wh_ref, bh_ref, o_ref):
    inv = 1.0 / jnp.maximum(c_ref[...][:, :1], 1.0)
    agg = jnp.concatenate([s0_ref[...], s1_ref[...]], axis=1) * inv
    hprev = jnp.concatenate([h0_ref[...], h1_ref[...]], axis=1)
    t = jnp.dot(agg, wl_ref[...], preferred_element_type=jnp.float32)
    t += jnp.dot(hprev, wr_ref[...], preferred_element_type=jnp.float32)
    t += b_ref[...]
    o_ref[...] = (jnp.dot(t, wh_ref[...], preferred_element_type=jnp.float32)
                  + bh_ref[...])


def _row_spec(width):
    return pl.BlockSpec((BLK, width), lambda i: (i, 0))


def _full_spec(shape):
    return pl.BlockSpec(shape, lambda i: (0,) * len(shape))


def _tc1(cnt, s0, s1, x, W1l, W1r, b1l):
    return pl.pallas_call(
        _tc1_body,
        grid=(N // BLK,),
        in_specs=[_row_spec(DH), _row_spec(DH), _row_spec(DH), _row_spec(D),
                  _full_spec((D, D)), _full_spec((D, D)), _full_spec((1, D))],
        out_specs=[_row_spec(DH), _row_spec(DH)],
        out_shape=[jax.ShapeDtypeStruct((N, DH), jnp.float32),
                   jax.ShapeDtypeStruct((N, DH), jnp.float32)],
    )(cnt, s0, s1, x, W1l, W1r, b1l)


def _tc2(cnt, s0, s1, h0, h1, W2l, W2r, b2l, Wh, bh):
    return pl.pallas_call(
        _tc2_body,
        grid=(N // BLK,),
        in_specs=[_row_spec(DH), _row_spec(DH), _row_spec(DH),
                  _row_spec(DH), _row_spec(DH),
                  _full_spec((D, D)), _full_spec((D, D)), _full_spec((1, D)),
                  _full_spec((D, D)), _full_spec((1, D))],
        out_specs=_row_spec(D),
        out_shape=jax.ShapeDtypeStruct((N, D), jnp.float32),
    )(cnt, s0, s1, h0, h1, W2l, W2r, b2l, Wh, bh)


def kernel(x, edge_index, W1l, b1l, W1r, W2l, b2l, W2r, Wh, bh):
    src4d = edge_index[0].reshape(NS, NGRP, GCHUNK, CHUNK)
    dst4d = edge_index[1].reshape(NS, NGRP, GCHUNK, CHUNK)
    xstack = jnp.concatenate([x[:, :DH], x[:, DH:]], axis=0)
    zrows = jnp.zeros((CHUNK, DH), jnp.float32)
    ones_h = jnp.ones((CHUNK, DH), jnp.float32)

    s_all, cnt_all = _agg_counts(xstack, src4d, dst4d, zrows, ones_h)
    s0 = s_all[:N]
    s1 = s_all[NPAD:NPAD + N]
    cnt = cnt_all[:N]
    h0, h1 = _tc1(cnt, s0, s1, x, W1l, W1r, b1l.reshape(1, D))
    hstack = jnp.concatenate([h0, h1], axis=0)
    t_all = _agg_plain(hstack, src4d, dst4d, zrows)
    t0 = t_all[:N]
    t1 = t_all[NPAD:NPAD + N]
    return _tc2(cnt, t0, t1, h0, h1, W2l, W2r, b2l.reshape(1, D),
                Wh, bh.reshape(1, D))


# speedup vs baseline: 5.4044x; 1.2135x over previous
import functools

import jax
import jax.numpy as jnp
from jax import lax
from jax.experimental import pallas as pl
from jax.experimental.pallas import tpu as pltpu
from jax.experimental.pallas import tpu_sc as plsc

N = 10000
NPAD = 10240
E = 160000
D = 256
DH = 128
NS = 16
CHUNK = 80
NGRP = 5
GCHUNK = 25
RPT = NPAD // NS
SLAB = 80
NSLAB = RPT // SLAB
LANES = 16


def _make_agg(with_counts):
    mesh = plsc.VectorSubcoreMesh(core_axis_name="c", subcore_axis_name="s")
    sums_ty = jax.ShapeDtypeStruct((2 * NPAD, DH), jnp.float32)
    out_type = ([sums_ty, jax.ShapeDtypeStruct((NPAD, DH), jnp.float32)]
                if with_counts else sums_ty)
    scratch = [
        pltpu.VMEM((GCHUNK, CHUNK), jnp.int32),
        pltpu.VMEM((GCHUNK, CHUNK), jnp.int32),
        pltpu.VMEM((CHUNK, DH), jnp.float32),
        pltpu.VMEM((CHUNK, DH), jnp.float32),
        pltpu.VMEM_SHARED((NPAD, DH), jnp.float32),
        pltpu.SemaphoreType.DMA,
        pltpu.SemaphoreType.DMA,
        pltpu.SemaphoreType.DMA,
        pltpu.SemaphoreType.DMA,
    ]

    def body(xstack, src4d, dst4d, zrows, *rest):
        if with_counts:
            (ones_h, out_s, out_c,
             src_v, dst_v, rows_a, rows_b, s_sp, gsa, gsb, ssa, ssb) = rest
        else:
            (out_s,
             src_v, dst_v, rows_a, rows_b, s_sp, gsa, gsb, ssa, ssb) = rest
        bufs = (rows_a, rows_b)
        gsems = (gsa, gsb)
        ssems = (ssa, ssb)
        cid = lax.axis_index("c")
        sid = lax.axis_index("s")
        base = sid * RPT
        coff = cid * N

        def zero_acc():
            pltpu.sync_copy(zrows, rows_a)
            ds = [pltpu.async_copy(
                rows_a, s_sp.at[pl.ds(base + j * SLAB, SLAB)], ssa)
                for j in range(NSLAB)]
            for d in ds:
                d.wait()

        def writeback(out_ref, row0):
            outd = [None, None]
            for j in range(NSLAB):
                b = j % 2
                if outd[b] is not None:
                    outd[b].wait()
                pltpu.async_copy(
                    s_sp.at[pl.ds(base + j * SLAB, SLAB)], bufs[b],
                    gsems[b]).wait()
                outd[b] = pltpu.async_copy(
                    bufs[b],
                    out_ref.at[pl.ds(row0 + base + j * SLAB, SLAB)],
                    ssems[b])
            for d in outd:
                if d is not None:
                    d.wait()

        zero_acc()
        plsc.subcore_barrier()

        def group(g, cg):
            pltpu.sync_copy(src4d.at[sid, g], src_v)
            pltpu.sync_copy(dst4d.at[sid, g], dst_v)

            def adjust(i, c):
                for k in range(CHUNK // LANES):
                    sl = pl.ds(k * LANES, LANES)
                    src_v[i, sl] = src_v[i, sl] + coff
                return c
            lax.fori_loop(0, GCHUNK, adjust, 0)

            gd = [None, None]
            sd = [None, None]
            gd[0] = pltpu.async_copy(xstack.at[src_v.at[0]], rows_a, gsa)
            for i in range(GCHUNK):
                b = i % 2
                gd[b].wait()
                if i + 1 < GCHUNK:
                    nb = (i + 1) % 2
                    if sd[nb] is not None:
                        sd[nb].wait()
                    gd[nb] = pltpu.async_copy(
                        xstack.at[src_v.at[i + 1]], bufs[nb], gsems[nb])
                sd[b] = pltpu.async_copy(
                    bufs[b], s_sp.at[dst_v.at[i]], ssems[b], add=True)
            if sd[(GCHUNK - 2) % 2] is not None:
                sd[(GCHUNK - 2) % 2].wait()
            sd[(GCHUNK - 1) % 2].wait()
            return cg
        lax.fori_loop(0, NGRP, group, 0)

        plsc.subcore_barrier()

        writeback(out_s, cid * NPAD)

        if with_counts:
            zero_acc()
            plsc.subcore_barrier()
            pltpu.sync_copy(ones_h, rows_b)

            def cgroup(g, cg):
                pltpu.sync_copy(dst4d.at[sid, g], dst_v)
                ds = [pltpu.async_copy(
                    rows_b, s_sp.at[dst_v.at[i]], ssb, add=True)
                    for i in range(GCHUNK)]
                for d in ds:
                    d.wait()
                return cg
            lax.fori_loop(0, NGRP, cgroup, 0)

            plsc.subcore_barrier()
            writeback(out_c, 0)

    return functools.partial(
        pl.kernel, mesh=mesh, out_type=out_type, scratch_types=scratch
    )(body)


_agg_counts = _make_agg(True)
_agg_plain = _make_agg(False)


BLK = 1000


def _tc1_body(c_ref, s0_ref, s1_ref, x_ref, wl_ref, wr_ref, b_ref,
              o0_ref, o1_ref):
    inv = 1.0 / jnp.maximum(c_ref[...][:, :1], 1.0)
    agg = jnp.concatenate([s0_ref[...], s1_ref[...]], axis=1) * inv
    h = jnp.dot(agg, wl_ref[...], preferred_element_type=jnp.float32)
    h += jnp.dot(x_ref[...], wr_ref[...], preferred_element_type=jnp.float32)
    h += b_ref[...]
    h = jnp.maximum(h, 0.0)
    o0_ref[...] = h[:, :DH]
    o1_ref[...] = h[:, DH:]


def _tc2_body(c_ref, s0_ref, s1_ref, h0_ref, h1_ref, wl_ref, wr_ref, b_ref,
              wh_ref, bh_ref, o_ref):
    inv = 1.0 / jnp.maximum(c_ref[...][:, :1], 1.0)
    agg = jnp.concatenate([s0_ref[...], s1_ref[...]], axis=1) * inv
    hprev = jnp.concatenate([h0_ref[...], h1_ref[...]], axis=1)
    t = jnp.dot(agg, wl_ref[...], preferred_element_type=jnp.float32)
    t += jnp.dot(hprev, wr_ref[...], preferred_element_type=jnp.float32)
    t += b_ref[...]
    o_ref[...] = (jnp.dot(t, wh_ref[...], preferred_element_type=jnp.float32)
                  + bh_ref[...])


def _row_spec(width):
    return pl.BlockSpec((BLK, width), lambda i: (i, 0))


def _full_spec(shape):
    return pl.BlockSpec(shape, lambda i: (0,) * len(shape))


def _tc1(cnt, s0, s1, x, W1l, W1r, b1l):
    return pl.pallas_call(
        _tc1_body,
        grid=(N // BLK,),
        in_specs=[_row_spec(DH), _row_spec(DH), _row_spec(DH), _row_spec(D),
                  _full_spec((D, D)), _full_spec((D, D)), _full_spec((1, D))],
        out_specs=[_row_spec(DH), _row_spec(DH)],
        out_shape=[jax.ShapeDtypeStruct((N, DH), jnp.float32),
                   jax.ShapeDtypeStruct((N, DH), jnp.float32)],
    )(cnt, s0, s1, x, W1l, W1r, b1l)


def _tc2(cnt, s0, s1, h0, h1, W2l, W2r, b2l, Wh, bh):
    return pl.pallas_call(
        _tc2_body,
        grid=(N // BLK,),
        in_specs=[_row_spec(DH), _row_spec(DH), _row_spec(DH),
                  _row_spec(DH), _row_spec(DH),
                  _full_spec((D, D)), _full_spec((D, D)), _full_spec((1, D)),
                  _full_spec((D, D)), _full_spec((1, D))],
        out_specs=_row_spec(D),
        out_shape=jax.ShapeDtypeStruct((N, D), jnp.float32),
    )(cnt, s0, s1, h0, h1, W2l, W2r, b2l, Wh, bh)


def kernel(x, edge_index, W1l, b1l, W1r, W2l, b2l, W2r, Wh, bh):
    src4d = edge_index[0].reshape(NS, NGRP, GCHUNK, CHUNK)
    dst4d = edge_index[1].reshape(NS, NGRP, GCHUNK, CHUNK)
    xstack = jnp.concatenate([x[:, :DH], x[:, DH:]], axis=0)
    zrows = jnp.zeros((CHUNK, DH), jnp.float32)
    ones_h = jnp.ones((CHUNK, DH), jnp.float32)

    s_all, cnt_all = _agg_counts(xstack, src4d, dst4d, zrows, ones_h)
    s0 = s_all[:N]
    s1 = s_all[NPAD:NPAD + N]
    cnt = cnt_all[:N]
    h0, h1 = _tc1(cnt, s0, s1, x, W1l, W1r, b1l.reshape(1, D))
    hstack = jnp.concatenate([h0, h1], axis=0)
    t_all = _agg_plain(hstack, src4d, dst4d, zrows)
    t0 = t_all[:N]
    t1 = t_all[NPAD:NPAD + N]
    return _tc2(cnt, t0, t1, h0, h1, W2l, W2r, b2l.reshape(1, D),
                Wh, bh.reshape(1, D))


# speedup vs baseline: 7.2762x; 1.3463x over previous
import functools

import jax
import jax.numpy as jnp
from jax import lax
from jax.experimental import pallas as pl
from jax.experimental.pallas import tpu as pltpu
from jax.experimental.pallas import tpu_sc as plsc

N = 10000
NPAD = 10240
E = 160000
D = 256
DH = 128
NS = 16
CHUNK = 80
NGRP = 5
GCHUNK = 25
RPT = NPAD // NS
SLAB = 80
NSLAB = RPT // SLAB
LANES = 16


def _make_agg(with_counts):
    mesh = plsc.VectorSubcoreMesh(core_axis_name="c", subcore_axis_name="s")
    sums_ty = jax.ShapeDtypeStruct((2 * NPAD, DH), jnp.float32)
    out_type = ([sums_ty, jax.ShapeDtypeStruct((2 * NPAD, DH), jnp.float32)]
                if with_counts else sums_ty)
    scratch = [
        pltpu.VMEM((GCHUNK, CHUNK), jnp.int32),
        pltpu.VMEM((GCHUNK, CHUNK), jnp.int32),
        pltpu.VMEM((CHUNK, DH), jnp.float32),
        pltpu.VMEM((CHUNK, DH), jnp.float32),
        pltpu.VMEM((CHUNK, DH), jnp.float32),
        pltpu.VMEM_SHARED((NPAD, DH), jnp.float32),
        pltpu.SemaphoreType.DMA,
        pltpu.SemaphoreType.DMA,
        pltpu.SemaphoreType.DMA,
        pltpu.SemaphoreType.DMA,
        pltpu.SemaphoreType.DMA,
        pltpu.SemaphoreType.DMA,
    ]
    NBUF = 3
    DEPTH_G = 2

    def body(xstack, src4d, dst4d, zrows, *rest):
        if with_counts:
            (ones_h, out_s, out_c, src_v, dst_v,
             rows_a, rows_b, rows_c, s_sp, gsa, gsb, gsc, ssa, ssb, ssc) = rest
        else:
            (out_s, src_v, dst_v,
             rows_a, rows_b, rows_c, s_sp, gsa, gsb, gsc, ssa, ssb, ssc) = rest
        bufs = (rows_a, rows_b, rows_c)
        gsems = (gsa, gsb, gsc)
        ssems = (ssa, ssb, ssc)
        cid = lax.axis_index("c")
        sid = lax.axis_index("s")
        base = sid * RPT
        coff = cid * N

        def zero_acc():
            pltpu.sync_copy(zrows, rows_a)
            ds = [pltpu.async_copy(
                rows_a, s_sp.at[pl.ds(base + j * SLAB, SLAB)], ssa)
                for j in range(NSLAB)]
            for d in ds:
                d.wait()

        def writeback(out_ref, row0):
            outd = [None, None]
            for j in range(NSLAB):
                b = j % 2
                if outd[b] is not None:
                    outd[b].wait()
                pltpu.async_copy(
                    s_sp.at[pl.ds(base + j * SLAB, SLAB)], bufs[b],
                    gsems[b]).wait()
                outd[b] = pltpu.async_copy(
                    bufs[b],
                    out_ref.at[pl.ds(row0 + base + j * SLAB, SLAB)],
                    ssems[b])
            for d in outd:
                if d is not None:
                    d.wait()

        zero_acc()
        plsc.subcore_barrier()

        def group(g, cg):
            pltpu.sync_copy(src4d.at[sid, g], src_v)
            pltpu.sync_copy(dst4d.at[sid, g], dst_v)

            def adjust(i, c):
                for k in range(CHUNK // LANES):
                    sl = pl.ds(k * LANES, LANES)
                    src_v[i, sl] = src_v[i, sl] + coff
                return c
            lax.fori_loop(0, GCHUNK, adjust, 0)

            gd = [None] * NBUF
            sd = [None] * NBUF
            for i in range(min(DEPTH_G, GCHUNK)):
                gd[i % NBUF] = pltpu.async_copy(
                    xstack.at[src_v.at[i]], bufs[i % NBUF], gsems[i % NBUF])
            for i in range(GCHUNK):
                b = i % NBUF
                gd[b].wait()
                j = i + DEPTH_G
                if j < GCHUNK:
                    nb = j % NBUF
                    if sd[nb] is not None:
                        sd[nb].wait()
                        sd[nb] = None
                    gd[nb] = pltpu.async_copy(
                        xstack.at[src_v.at[j]], bufs[nb], gsems[nb])
                sd[b] = pltpu.async_copy(
                    bufs[b], s_sp.at[dst_v.at[i]], ssems[b], add=True)
            for d in sd:
                if d is not None:
                    d.wait()
            return cg
        lax.fori_loop(0, NGRP, group, 0)

        plsc.subcore_barrier()

        writeback(out_s, cid * NPAD)

        if with_counts:
            zero_acc()
            plsc.subcore_barrier()
            pltpu.sync_copy(ones_h, rows_b)

            def cgroup(g, cg):
                pltpu.sync_copy(dst4d.at[sid, g], dst_v)
                ds = [pltpu.async_copy(
                    rows_b, s_sp.at[dst_v.at[i]], ssb, add=True)
                    for i in range(GCHUNK)]
                for d in ds:
                    d.wait()
                return cg
            lax.fori_loop(cid * 3, 3 + cid * 2, cgroup, 0)

            plsc.subcore_barrier()
            writeback(out_c, cid * NPAD)

    return functools.partial(
        pl.kernel, mesh=mesh, out_type=out_type, scratch_types=scratch
    )(body)


_agg_counts = _make_agg(True)
_agg_plain = _make_agg(False)


BLK = 1000


def _tc1_body(c0_ref, c1_ref, s0_ref, s1_ref, x_ref, wl_ref, wr_ref, b_ref,
              o0_ref, o1_ref):
    cnt = c0_ref[...][:, :1] + c1_ref[...][:, :1]
    inv = 1.0 / jnp.maximum(cnt, 1.0)
    agg = jnp.concatenate([s0_ref[...], s1_ref[...]], axis=1) * inv
    h = jnp.dot(agg, wl_ref[...], preferred_element_type=jnp.float32)
    h += jnp.dot(x_ref[...], wr_ref[...], preferred_element_type=jnp.float32)
    h += b_ref[...]
    h = jnp.maximum(h, 0.0)
    o0_ref[...] = h[:, :DH]
    o1_ref[...] = h[:, DH:]


def _tc2_body(c0_ref, c1_ref, s0_ref, s1_ref, h0_ref, h1_ref,
              wl_ref, wr_ref, b_ref, wh_ref, bh_ref, o_ref):
    cnt = c0_ref[...][:, :1] + c1_ref[...][:, :1]
    inv = 1.0 / jnp.maximum(cnt, 1.0)
    agg = jnp.concatenate([s0_ref[...], s1_ref[...]], axis=1) * inv
    hprev = jnp.concatenate([h0_ref[...], h1_ref[...]], axis=1)
    t = jnp.dot(agg, wl_ref[...], preferred_element_type=jnp.float32)
    t += jnp.dot(hprev, wr_ref[...], preferred_element_type=jnp.float32)
    t += b_ref[...]
    o_ref[...] = (jnp.dot(t, wh_ref[...], preferred_element_type=jnp.float32)
                  + bh_ref[...])


def _row_spec(width):
    return pl.BlockSpec((BLK, width), lambda i: (i, 0))


def _full_spec(shape):
    return pl.BlockSpec(shape, lambda i: (0,) * len(shape))


def _tc1(c0, c1, s0, s1, x, W1l, W1r, b1l):
    return pl.pallas_call(
        _tc1_body,
        grid=(N // BLK,),
        in_specs=[_row_spec(DH), _row_spec(DH),
                  _row_spec(DH), _row_spec(DH), _row_spec(D),
                  _full_spec((D, D)), _full_spec((D, D)), _full_spec((1, D))],
        out_specs=[_row_spec(DH), _row_spec(DH)],
        out_shape=[jax.ShapeDtypeStruct((N, DH), jnp.float32),
                   jax.ShapeDtypeStruct((N, DH), jnp.float32)],
    )(c0, c1, s0, s1, x, W1l, W1r, b1l)


def _tc2(c0, c1, s0, s1, h0, h1, W2l, W2r, b2l, Wh, bh):
    return pl.pallas_call(
        _tc2_body,
        grid=(N // BLK,),
        in_specs=[_row_spec(DH), _row_spec(DH), _row_spec(DH), _row_spec(DH),
                  _row_spec(DH), _row_spec(DH),
                  _full_spec((D, D)), _full_spec((D, D)), _full_spec((1, D)),
                  _full_spec((D, D)), _full_spec((1, D))],
        out_specs=_row_spec(D),
        out_shape=jax.ShapeDtypeStruct((N, D), jnp.float32),
    )(c0, c1, s0, s1, h0, h1, W2l, W2r, b2l, Wh, bh)


def kernel(x, edge_index, W1l, b1l, W1r, W2l, b2l, W2r, Wh, bh):
    src4d = edge_index[0].reshape(NS, NGRP, GCHUNK, CHUNK)
    dst4d = edge_index[1].reshape(NS, NGRP, GCHUNK, CHUNK)
    xstack = jnp.concatenate([x[:, :DH], x[:, DH:]], axis=0)
    zrows = jnp.zeros((CHUNK, DH), jnp.float32)
    ones_h = jnp.ones((CHUNK, DH), jnp.float32)

    s_all, cnt_all = _agg_counts(xstack, src4d, dst4d, zrows, ones_h)
    s0 = s_all[:N]
    s1 = s_all[NPAD:NPAD + N]
    c0 = cnt_all[:N]
    c1 = cnt_all[NPAD:NPAD + N]
    h0, h1 = _tc1(c0, c1, s0, s1, x, W1l, W1r, b1l.reshape(1, D))
    hstack = jnp.concatenate([h0, h1], axis=0)
    t_all = _agg_plain(hstack, src4d, dst4d, zrows)
    t0 = t_all[:N]
    t1 = t_all[NPAD:NPAD + N]
    return _tc2(c0, c1, t0, t1, h0, h1, W2l, W2r, b2l.reshape(1, D),
                Wh, bh.reshape(1, D))
